# baseline (device time: 28711 ns/iter reference)
import jax
import jax.numpy as jnp
from jax import lax
from jax.experimental import pallas as pl
from jax.experimental.pallas import tpu as pltpu

T = 512
D = 1024
V_SHARD = 8192
VB = 1024
N_STEPS = V_SHARD // VB


def kernel(x, W, labels):
    labels2d = labels.reshape(T, 1)

    def body(x_ref, w_ref, lab_ref, out_ref, stats_ref, recv_ref, send_sem, recv_sem):
        j = pl.program_id(0)

        logits = jnp.dot(x_ref[:, :], w_ref[:, :], preferred_element_type=jnp.float32)
        e = jnp.exp(logits)
        col = lax.broadcasted_iota(jnp.int32, (T, VB), 1)
        rel = lab_ref[:, :] - lax.axis_index("x") * V_SHARD - j * VB
        sel = jnp.where(col == rel, logits, 0.0)
        ones = jnp.ones((VB, 1), jnp.float32)
        s_chunk = jnp.dot(e, ones, preferred_element_type=jnp.float32)[:, 0]
        l_chunk = jnp.dot(sel, ones, preferred_element_type=jnp.float32)[:, 0]

        @pl.when(j == 0)
        def _():
            stats_ref[0, :] = s_chunk
            stats_ref[1, :] = l_chunk

        @pl.when(j > 0)
        def _():
            stats_ref[0, :] = stats_ref[0, :] + s_chunk
            stats_ref[1, :] = stats_ref[1, :] + l_chunk

        @pl.when(j == N_STEPS - 1)
        def _():
            partner = (
                1 - lax.axis_index("x"),
                lax.axis_index("y"),
                lax.axis_index("z"),
            )

            barrier_sem = pltpu.get_barrier_semaphore()
            pl.semaphore_signal(
                barrier_sem, inc=1, device_id=partner,
                device_id_type=pl.DeviceIdType.MESH,
            )
            pl.semaphore_wait(barrier_sem, 1)

            rdma = pltpu.make_async_remote_copy(
                src_ref=stats_ref,
                dst_ref=recv_ref,
                send_sem=send_sem,
                recv_sem=recv_sem,
                device_id=partner,
                device_id_type=pl.DeviceIdType.MESH,
            )
            rdma.start()
            rdma.wait()

            s = stats_ref[0, :] + recv_ref[0, :]
            l = stats_ref[1, :] + recv_ref[1, :]
            out_ref[:] = jnp.log(s) - l

    return pl.pallas_call(
        body,
        grid=(N_STEPS,),
        out_shape=jax.ShapeDtypeStruct((T,), jnp.float32),
        in_specs=[
            pl.BlockSpec((T, D), lambda j: (0, 0), memory_space=pltpu.VMEM),
            pl.BlockSpec((D, VB), lambda j: (0, j), memory_space=pltpu.VMEM),
            pl.BlockSpec((T, 1), lambda j: (0, 0), memory_space=pltpu.VMEM),
        ],
        out_specs=pl.BlockSpec((T,), lambda j: (0,), memory_space=pltpu.VMEM),
        scratch_shapes=[
            pltpu.VMEM((2, T), jnp.float32),
            pltpu.VMEM((2, T), jnp.float32),
            pltpu.SemaphoreType.DMA,
            pltpu.SemaphoreType.DMA,
        ],
        compiler_params=pltpu.CompilerParams(
            collective_id=0,
            dimension_semantics=("arbitrary",),
        ),
    )(x, W, labels2d)


# device time: 24478 ns/iter; 1.1729x vs baseline; 1.1729x over previous
import jax
import jax.numpy as jnp
from jax import lax
from jax.experimental import pallas as pl
from jax.experimental.pallas import tpu as pltpu

T = 512
D = 1024
V_SHARD = 8192
VB = 2048
N_STEPS = V_SHARD // VB


def kernel(x, W, labels):
    labels2d = labels.reshape(T, 1)

    def body(x_ref, w_ref, lab_ref, out_ref, stats_ref, recv_ref, send_sem, recv_sem):
        j = pl.program_id(0)

        logits = jnp.dot(x_ref[:, :], w_ref[:, :], preferred_element_type=jnp.float32)
        s_chunk = jnp.sum(jnp.exp(logits), axis=1)
        col = lax.broadcasted_iota(jnp.int32, (T, VB), 1)
        rel = lab_ref[:, :] - lax.axis_index("x") * V_SHARD - j * VB
        l_chunk = jnp.sum(jnp.where(col == rel, logits, 0.0), axis=1)

        @pl.when(j == 0)
        def _():
            stats_ref[0, :] = s_chunk
            stats_ref[1, :] = l_chunk

        @pl.when(j > 0)
        def _():
            stats_ref[0, :] = stats_ref[0, :] + s_chunk
            stats_ref[1, :] = stats_ref[1, :] + l_chunk

        @pl.when(j == N_STEPS - 1)
        def _():
            partner = (
                1 - lax.axis_index("x"),
                lax.axis_index("y"),
                lax.axis_index("z"),
            )

            barrier_sem = pltpu.get_barrier_semaphore()
            pl.semaphore_signal(
                barrier_sem, inc=1, device_id=partner,
                device_id_type=pl.DeviceIdType.MESH,
            )
            pl.semaphore_wait(barrier_sem, 1)

            rdma = pltpu.make_async_remote_copy(
                src_ref=stats_ref,
                dst_ref=recv_ref,
                send_sem=send_sem,
                recv_sem=recv_sem,
                device_id=partner,
                device_id_type=pl.DeviceIdType.MESH,
            )
            rdma.start()
            rdma.wait()

            s = stats_ref[0, :] + recv_ref[0, :]
            l = stats_ref[1, :] + recv_ref[1, :]
            out_ref[:] = jnp.log(s) - l

    return pl.pallas_call(
        body,
        grid=(N_STEPS,),
        out_shape=jax.ShapeDtypeStruct((T,), jnp.float32),
        in_specs=[
            pl.BlockSpec((T, D), lambda j: (0, 0), memory_space=pltpu.VMEM),
            pl.BlockSpec((D, VB), lambda j: (0, j), memory_space=pltpu.VMEM),
            pl.BlockSpec((T, 1), lambda j: (0, 0), memory_space=pltpu.VMEM),
        ],
        out_specs=pl.BlockSpec((T,), lambda j: (0,), memory_space=pltpu.VMEM),
        scratch_shapes=[
            pltpu.VMEM((2, T), jnp.float32),
            pltpu.VMEM((2, T), jnp.float32),
            pltpu.SemaphoreType.DMA,
            pltpu.SemaphoreType.DMA,
        ],
        compiler_params=pltpu.CompilerParams(
            collective_id=0,
            dimension_semantics=("arbitrary",),
        ),
    )(x, W, labels2d)
